# baseline (speedup 1.0000x reference)
import functools

import jax
import jax.numpy as jnp
import numpy as np
from jax import lax
from jax.experimental import pallas as pl
from jax.experimental.pallas import tpu as pltpu
from jax.experimental.pallas import tpu_sc as plsc

_NUM_ITEMS = 100000
_B = 4096
_M = 2 * _B
_SEG = _M // 16


def _threefry_uniform_f32(seed: int, size: int) -> np.ndarray:
    def rotl(x, d):
        return ((x << np.uint32(d)) | (x >> np.uint32(32 - d))).astype(np.uint32)

    i64 = np.arange(size, dtype=np.uint64)
    x0 = (i64 >> np.uint64(32)).astype(np.uint32)
    x1 = (i64 & np.uint64(0xFFFFFFFF)).astype(np.uint32)
    k0 = np.uint32(seed >> 32)
    k1 = np.uint32(seed & 0xFFFFFFFF)
    ks = [k0, k1, np.uint32(k0 ^ k1 ^ np.uint32(0x1BD11BDA))]
    rotations = [(13, 15, 26, 6), (17, 29, 16, 24)]
    x0 = (x0 + ks[0]).astype(np.uint32)
    x1 = (x1 + ks[1]).astype(np.uint32)
    for i in range(5):
        for r in rotations[i % 2]:
            x0 = (x0 + x1).astype(np.uint32)
            x1 = rotl(x1, r)
            x1 = (x1 ^ x0).astype(np.uint32)
        x0 = (x0 + ks[(i + 1) % 3]).astype(np.uint32)
        x1 = (x1 + ks[(i + 2) % 3] + np.uint32(i + 1)).astype(np.uint32)
    bits = (x0 ^ x1).astype(np.uint32)
    mant = (bits >> np.uint32(9)) | np.uint32(0x3F800000)
    return mant.view(np.float32) - np.float32(1.0)


_scores = _threefry_uniform_f32(42, _NUM_ITEMS)
_order = np.argsort(-_scores, kind="stable").astype(np.int32)
_rank_np = np.empty((_NUM_ITEMS,), dtype=np.int32)
_rank_np[_order] = np.arange(_NUM_ITEMS, dtype=np.int32)
_RANK = _rank_np
_PREF_T = np.ascontiguousarray(
    _order[:_M].reshape(16, _SEG).T.reshape(-1))

_mesh = plsc.VectorSubcoreMesh(core_axis_name="c", subcore_axis_name="s")


@functools.partial(
    pl.kernel,
    out_type=jax.ShapeDtypeStruct((_B,), jnp.int32),
    mesh=_mesh,
    scratch_types=[
        pltpu.VMEM((_B,), jnp.int32),
        pltpu.VMEM((_NUM_ITEMS,), jnp.int32),
        pltpu.VMEM((_M,), jnp.int32),
        pltpu.VMEM((_M + 256,), jnp.int32),
        pltpu.VMEM((_B + 16,), jnp.int32),
        pltpu.SemaphoreType.DMA,
        pltpu.SemaphoreType.DMA,
        pltpu.SemaphoreType.DMA,
    ],
    compiler_params=pltpu.CompilerParams(needs_layout_passes=False),
)
def _negatives(item_hbm, rank_hbm, pref_hbm, out_hbm,
               it_v, rank_v, p_v, mask_v, out_v, s_a, s_b, s_c):
    c = lax.axis_index("c")
    s = lax.axis_index("s")

    @pl.when(jnp.logical_and(c == 0, s == 0))
    def _():
        cp_it = pltpu.async_copy(item_hbm, it_v, s_a)
        cp_rk = pltpu.async_copy(rank_hbm, rank_v, s_b)
        cp_p = pltpu.async_copy(pref_hbm, p_v, s_c)

        zeros16 = jnp.zeros((16,), jnp.int32)
        ones16 = jnp.ones((16,), jnp.int32)

        @plsc.parallel_loop(0, (_M + 256) // 16, unroll=8)
        def _zero(i):
            mask_v[pl.ds(16 * i, 16)] = zeros16

        cp_it.wait()
        cp_rk.wait()

        @plsc.parallel_loop(0, _B // 16, unroll=8)
        def _mark(i):
            idx = it_v[pl.ds(16 * i, 16)]
            r = plsc.load_gather(rank_v, [idx])
            rt = jnp.where(r < _M,
                           16 * (r & (_SEG - 1)) + (r >> 9), _M)
            plsc.store_scatter(mask_v, [rt], ones16)

        @plsc.parallel_loop(0, _SEG, unroll=8,
                            carry=jnp.zeros((16,), jnp.int32))
        def _count(i, acc):
            m = mask_v[pl.ds(16 * i, 16)]
            return acc + jnp.where(m == 0, 1, 0)

        lane_cnt = _count
        lane_base = plsc.cumsum(lane_cnt) - lane_cnt

        cp_p.wait()

        @plsc.parallel_loop(0, _SEG, unroll=8, carry=lane_base)
        def _compact(i, run):
            m = mask_v[pl.ds(16 * i, 16)]
            pval = p_v[pl.ds(16 * i, 16)]
            keep = jnp.where(m == 0, 1, 0)
            valid = jnp.logical_and(m == 0, run < _B)
            dst = jnp.where(valid, run, _B)
            plsc.store_scatter(out_v, [dst], pval)
            return run + keep

        pltpu.sync_copy(out_v.at[pl.ds(0, _B)], out_hbm)


def kernel(user, item, target):
    negative_item = _negatives(item.astype(jnp.int32), _RANK,
                               _PREF_T).astype(item.dtype)
    user_out = jnp.full((_B + _B,), user[0], dtype=user.dtype)
    item_out = jnp.concatenate([item, negative_item], axis=0)
    target_out = jnp.concatenate(
        [target, jnp.zeros((_B,), dtype=target.dtype)], axis=0)
    return (user_out, item_out, target_out)


# speedup vs baseline: 1.0583x; 1.0583x over previous
import functools

import jax
import jax.numpy as jnp
import numpy as np
from jax import lax
from jax.experimental import pallas as pl
from jax.experimental.pallas import tpu as pltpu
from jax.experimental.pallas import tpu_sc as plsc

_NUM_ITEMS = 100000
_B = 4096
_M = 2 * _B
_SEG = _M // 16


def _threefry_uniform_f32(seed: int, size: int) -> np.ndarray:
    def rotl(x, d):
        return ((x << np.uint32(d)) | (x >> np.uint32(32 - d))).astype(np.uint32)

    i64 = np.arange(size, dtype=np.uint64)
    x0 = (i64 >> np.uint64(32)).astype(np.uint32)
    x1 = (i64 & np.uint64(0xFFFFFFFF)).astype(np.uint32)
    k0 = np.uint32(seed >> 32)
    k1 = np.uint32(seed & 0xFFFFFFFF)
    ks = [k0, k1, np.uint32(k0 ^ k1 ^ np.uint32(0x1BD11BDA))]
    rotations = [(13, 15, 26, 6), (17, 29, 16, 24)]
    x0 = (x0 + ks[0]).astype(np.uint32)
    x1 = (x1 + ks[1]).astype(np.uint32)
    for i in range(5):
        for r in rotations[i % 2]:
            x0 = (x0 + x1).astype(np.uint32)
            x1 = rotl(x1, r)
            x1 = (x1 ^ x0).astype(np.uint32)
        x0 = (x0 + ks[(i + 1) % 3]).astype(np.uint32)
        x1 = (x1 + ks[(i + 2) % 3] + np.uint32(i + 1)).astype(np.uint32)
    bits = (x0 ^ x1).astype(np.uint32)
    mant = (bits >> np.uint32(9)) | np.uint32(0x3F800000)
    return mant.view(np.float32) - np.float32(1.0)


_scores = _threefry_uniform_f32(42, _NUM_ITEMS)
_order = np.argsort(-_scores, kind="stable").astype(np.int32)
_rank_np = np.empty((_NUM_ITEMS,), dtype=np.int32)
_rank_np[_order] = np.arange(_NUM_ITEMS, dtype=np.int32)
_RANK = _rank_np
_PREF_T = np.ascontiguousarray(
    _order[:_M].reshape(16, _SEG).T.reshape(-1))

_mesh = plsc.VectorSubcoreMesh(core_axis_name="c", subcore_axis_name="s",
                               num_cores=1)


@functools.partial(
    pl.kernel,
    out_type=jax.ShapeDtypeStruct((_B,), jnp.int32),
    mesh=_mesh,
    scratch_types=[
        pltpu.VMEM((_B,), jnp.int32),
        pltpu.VMEM((_NUM_ITEMS,), jnp.int32),
        pltpu.VMEM((_M,), jnp.int32),
        pltpu.VMEM((_M + 256,), jnp.int32),
        pltpu.VMEM((_B + 16,), jnp.int32),
        pltpu.SemaphoreType.DMA,
        pltpu.SemaphoreType.DMA,
        pltpu.SemaphoreType.DMA,
    ],
    compiler_params=pltpu.CompilerParams(needs_layout_passes=False),
)
def _negatives(item_hbm, rank_hbm, pref_hbm, out_hbm,
               it_v, rank_v, p_v, mask_v, out_v, s_a, s_b, s_c):
    c = lax.axis_index("c")
    s = lax.axis_index("s")

    @pl.when(jnp.logical_and(c == 0, s == 0))
    def _():
        cp_it = pltpu.async_copy(item_hbm, it_v, s_a)
        cp_rk = pltpu.async_copy(rank_hbm, rank_v, s_b)
        cp_p = pltpu.async_copy(pref_hbm, p_v, s_c)

        zeros16 = jnp.zeros((16,), jnp.int32)
        ones16 = jnp.ones((16,), jnp.int32)

        @plsc.parallel_loop(0, (_M + 256) // 16, unroll=8)
        def _zero(i):
            mask_v[pl.ds(16 * i, 16)] = zeros16

        cp_it.wait()
        cp_rk.wait()

        @plsc.parallel_loop(0, _B // 16, unroll=8)
        def _mark(i):
            idx = it_v[pl.ds(16 * i, 16)]
            r = plsc.load_gather(rank_v, [idx])
            rt = jnp.where(r < _M,
                           16 * (r & (_SEG - 1)) + (r >> 9), _M)
            plsc.store_scatter(mask_v, [rt], ones16)

        @plsc.parallel_loop(0, _SEG, unroll=8,
                            carry=jnp.zeros((16,), jnp.int32))
        def _count(i, acc):
            m = mask_v[pl.ds(16 * i, 16)]
            return acc + jnp.where(m == 0, 1, 0)

        lane_cnt = _count
        lane_base = plsc.cumsum(lane_cnt) - lane_cnt

        cp_p.wait()

        @plsc.parallel_loop(0, _SEG, unroll=8, carry=lane_base)
        def _compact(i, run):
            m = mask_v[pl.ds(16 * i, 16)]
            pval = p_v[pl.ds(16 * i, 16)]
            keep = jnp.where(m == 0, 1, 0)
            valid = jnp.logical_and(m == 0, run < _B)
            dst = jnp.where(valid, run, _B)
            plsc.store_scatter(out_v, [dst], pval)
            return run + keep

        pltpu.sync_copy(out_v.at[pl.ds(0, _B)], out_hbm)


def kernel(user, item, target):
    negative_item = _negatives(item.astype(jnp.int32), _RANK,
                               _PREF_T).astype(item.dtype)
    user_out = jnp.full((_B + _B,), user[0], dtype=user.dtype)
    item_out = jnp.concatenate([item, negative_item], axis=0)
    target_out = jnp.concatenate(
        [target, jnp.zeros((_B,), dtype=target.dtype)], axis=0)
    return (user_out, item_out, target_out)


# speedup vs baseline: 1.1398x; 1.0769x over previous
import functools

import jax
import jax.numpy as jnp
import numpy as np
from jax import lax
from jax.experimental import pallas as pl
from jax.experimental.pallas import tpu as pltpu
from jax.experimental.pallas import tpu_sc as plsc

_NUM_ITEMS = 100000
_B = 4096
_M = 2 * _B
_SEG = _M // 16


def _threefry_uniform_f32(seed: int, size: int) -> np.ndarray:
    def rotl(x, d):
        return ((x << np.uint32(d)) | (x >> np.uint32(32 - d))).astype(np.uint32)

    i64 = np.arange(size, dtype=np.uint64)
    x0 = (i64 >> np.uint64(32)).astype(np.uint32)
    x1 = (i64 & np.uint64(0xFFFFFFFF)).astype(np.uint32)
    k0 = np.uint32(seed >> 32)
    k1 = np.uint32(seed & 0xFFFFFFFF)
    ks = [k0, k1, np.uint32(k0 ^ k1 ^ np.uint32(0x1BD11BDA))]
    rotations = [(13, 15, 26, 6), (17, 29, 16, 24)]
    x0 = (x0 + ks[0]).astype(np.uint32)
    x1 = (x1 + ks[1]).astype(np.uint32)
    for i in range(5):
        for r in rotations[i % 2]:
            x0 = (x0 + x1).astype(np.uint32)
            x1 = rotl(x1, r)
            x1 = (x1 ^ x0).astype(np.uint32)
        x0 = (x0 + ks[(i + 1) % 3]).astype(np.uint32)
        x1 = (x1 + ks[(i + 2) % 3] + np.uint32(i + 1)).astype(np.uint32)
    bits = (x0 ^ x1).astype(np.uint32)
    mant = (bits >> np.uint32(9)) | np.uint32(0x3F800000)
    return mant.view(np.float32) - np.float32(1.0)


_scores = _threefry_uniform_f32(42, _NUM_ITEMS)
_order = np.argsort(-_scores, kind="stable").astype(np.int32)
_rank_np = np.empty((_NUM_ITEMS,), dtype=np.int32)
_rank_np[_order] = np.arange(_NUM_ITEMS, dtype=np.int32)
_slot_np = np.where(_rank_np < _M,
                    16 * (_rank_np & (_SEG - 1)) + (_rank_np >> 9),
                    _M).astype(np.uint32)
_SLOT2 = (_slot_np[0::2] | (_slot_np[1::2] << 16)).astype(np.int32)
_PREF_T = np.ascontiguousarray(
    _order[:_M].reshape(16, _SEG).T.reshape(-1))

_mesh = plsc.VectorSubcoreMesh(core_axis_name="c", subcore_axis_name="s",
                               num_cores=1)


@functools.partial(
    pl.kernel,
    out_type=jax.ShapeDtypeStruct((_B,), jnp.int32),
    mesh=_mesh,
    scratch_types=[
        pltpu.VMEM((_B,), jnp.int32),
        pltpu.VMEM((_NUM_ITEMS // 2,), jnp.int32),
        pltpu.VMEM((_M,), jnp.int32),
        pltpu.VMEM((_M + 256,), jnp.int32),
        pltpu.VMEM((_B + 16,), jnp.int32),
        pltpu.SemaphoreType.DMA,
        pltpu.SemaphoreType.DMA,
        pltpu.SemaphoreType.DMA,
    ],
    compiler_params=pltpu.CompilerParams(needs_layout_passes=False),
)
def _negatives(item_hbm, rank_hbm, pref_hbm, out_hbm,
               it_v, rank_v, p_v, mask_v, out_v, s_a, s_b, s_c):
    c = lax.axis_index("c")
    s = lax.axis_index("s")

    @pl.when(jnp.logical_and(c == 0, s == 0))
    def _():
        cp_it = pltpu.async_copy(item_hbm, it_v, s_a)
        cp_rk = pltpu.async_copy(rank_hbm, rank_v, s_b)
        cp_p = pltpu.async_copy(pref_hbm, p_v, s_c)

        zeros16 = jnp.zeros((16,), jnp.int32)
        ones16 = jnp.ones((16,), jnp.int32)

        @plsc.parallel_loop(0, (_M + 256) // 16, unroll=8)
        def _zero(i):
            mask_v[pl.ds(16 * i, 16)] = zeros16

        cp_it.wait()
        cp_rk.wait()

        @plsc.parallel_loop(0, _B // 16, unroll=8)
        def _mark(i):
            idx = it_v[pl.ds(16 * i, 16)]
            word = plsc.load_gather(rank_v, [idx >> 1])
            rt = (word >> ((idx & 1) << 4)) & 0xFFFF
            plsc.store_scatter(mask_v, [rt], ones16)

        @plsc.parallel_loop(0, _SEG, unroll=8,
                            carry=jnp.zeros((16,), jnp.int32))
        def _count(i, acc):
            m = mask_v[pl.ds(16 * i, 16)]
            return acc + jnp.where(m == 0, 1, 0)

        lane_cnt = _count
        lane_base = plsc.cumsum(lane_cnt) - lane_cnt

        cp_p.wait()

        @plsc.parallel_loop(0, _SEG, unroll=8, carry=lane_base)
        def _compact(i, run):
            m = mask_v[pl.ds(16 * i, 16)]
            pval = p_v[pl.ds(16 * i, 16)]
            keep = jnp.where(m == 0, 1, 0)
            valid = jnp.logical_and(m == 0, run < _B)
            dst = jnp.where(valid, run, _B)
            plsc.store_scatter(out_v, [dst], pval)
            return run + keep

        pltpu.sync_copy(out_v.at[pl.ds(0, _B)], out_hbm)


def kernel(user, item, target):
    negative_item = _negatives(item.astype(jnp.int32), _SLOT2,
                               _PREF_T).astype(item.dtype)
    user_out = jnp.full((_B + _B,), user[0], dtype=user.dtype)
    item_out = jnp.concatenate([item, negative_item], axis=0)
    target_out = jnp.concatenate(
        [target, jnp.zeros((_B,), dtype=target.dtype)], axis=0)
    return (user_out, item_out, target_out)


# speedup vs baseline: 1.2045x; 1.0568x over previous
import functools

import jax
import jax.numpy as jnp
import numpy as np
from jax import lax
from jax.experimental import pallas as pl
from jax.experimental.pallas import tpu as pltpu
from jax.experimental.pallas import tpu_sc as plsc

_NUM_ITEMS = 100000
_B = 4096
_M = 2 * _B
_SEG = _M // 16


def _threefry_uniform_f32(seed: int, size: int) -> np.ndarray:
    def rotl(x, d):
        return ((x << np.uint32(d)) | (x >> np.uint32(32 - d))).astype(np.uint32)

    i64 = np.arange(size, dtype=np.uint64)
    x0 = (i64 >> np.uint64(32)).astype(np.uint32)
    x1 = (i64 & np.uint64(0xFFFFFFFF)).astype(np.uint32)
    k0 = np.uint32(seed >> 32)
    k1 = np.uint32(seed & 0xFFFFFFFF)
    ks = [k0, k1, np.uint32(k0 ^ k1 ^ np.uint32(0x1BD11BDA))]
    rotations = [(13, 15, 26, 6), (17, 29, 16, 24)]
    x0 = (x0 + ks[0]).astype(np.uint32)
    x1 = (x1 + ks[1]).astype(np.uint32)
    for i in range(5):
        for r in rotations[i % 2]:
            x0 = (x0 + x1).astype(np.uint32)
            x1 = rotl(x1, r)
            x1 = (x1 ^ x0).astype(np.uint32)
        x0 = (x0 + ks[(i + 1) % 3]).astype(np.uint32)
        x1 = (x1 + ks[(i + 2) % 3] + np.uint32(i + 1)).astype(np.uint32)
    bits = (x0 ^ x1).astype(np.uint32)
    mant = (bits >> np.uint32(9)) | np.uint32(0x3F800000)
    return mant.view(np.float32) - np.float32(1.0)


_scores = _threefry_uniform_f32(42, _NUM_ITEMS)
_order = np.argsort(-_scores, kind="stable").astype(np.int32)
_rank_np = np.empty((_NUM_ITEMS,), dtype=np.int32)
_rank_np[_order] = np.arange(_NUM_ITEMS, dtype=np.int32)
_slot_np = np.where(_rank_np < _M,
                    16 * (_rank_np & (_SEG - 1)) + (_rank_np >> 9),
                    _M).astype(np.uint32)
_SLOT2 = (_slot_np[0::2] | (_slot_np[1::2] << 16)).astype(np.int32)
_PREF_T = np.ascontiguousarray(
    _order[:_M].reshape(16, _SEG).T.reshape(-1))

_mesh = plsc.VectorSubcoreMesh(core_axis_name="c", subcore_axis_name="s",
                               num_cores=1)


@functools.partial(
    pl.kernel,
    out_type=(
        jax.ShapeDtypeStruct((2 * _B,), jnp.int32),
        jax.ShapeDtypeStruct((2 * _B,), jnp.int32),
        jax.ShapeDtypeStruct((2 * _B,), jnp.float32),
    ),
    mesh=_mesh,
    scratch_types=[
        pltpu.VMEM((_B,), jnp.int32),
        pltpu.VMEM((_NUM_ITEMS // 2,), jnp.int32),
        pltpu.VMEM((_M,), jnp.int32),
        pltpu.VMEM((_M + 256,), jnp.int32),
        pltpu.VMEM((_B + 16,), jnp.int32),
        pltpu.VMEM((16,), jnp.int32),
        pltpu.VMEM((2 * _B,), jnp.int32),
        pltpu.VMEM((2 * _B,), jnp.float32),
        pltpu.SemaphoreType.DMA,
        pltpu.SemaphoreType.DMA,
        pltpu.SemaphoreType.DMA,
        pltpu.SemaphoreType.DMA,
        pltpu.SemaphoreType.DMA,
    ],
    compiler_params=pltpu.CompilerParams(needs_layout_passes=False),
)
def _negatives(user_hbm, item_hbm, target_hbm, slot_hbm, pref_hbm,
               uo_hbm, io_hbm, to_hbm,
               it_v, tbl_v, p_v, mask_v, out_v, u_v, uf_v, tg_v,
               s_a, s_b, s_c, s_d, s_e):
    c = lax.axis_index("c")
    s = lax.axis_index("s")

    @pl.when(jnp.logical_and(c == 0, s == 0))
    def _():
        cp_it = pltpu.async_copy(item_hbm, it_v, s_a)
        cp_tb = pltpu.async_copy(slot_hbm, tbl_v, s_b)
        cp_p = pltpu.async_copy(pref_hbm, p_v, s_c)
        cp_u = pltpu.async_copy(user_hbm.at[pl.ds(0, 16)], u_v, s_d)
        cp_tg = pltpu.async_copy(target_hbm, tg_v.at[pl.ds(0, _B)], s_e)

        zeros16 = jnp.zeros((16,), jnp.int32)
        ones16 = jnp.ones((16,), jnp.int32)
        fzeros16 = jnp.zeros((16,), jnp.float32)

        @plsc.parallel_loop(0, _B // 16, unroll=8)
        def _tzero(i):
            tg_v[pl.ds(_B + 16 * i, 16)] = fzeros16

        @plsc.parallel_loop(0, (_M + 256) // 16, unroll=8)
        def _zero(i):
            mask_v[pl.ds(16 * i, 16)] = zeros16

        cp_it.wait()
        cp_tb.wait()

        @plsc.parallel_loop(0, _B // 16, unroll=8)
        def _mark(i):
            idx = it_v[pl.ds(16 * i, 16)]
            word = plsc.load_gather(tbl_v, [idx >> 1])
            rt = (word >> ((idx & 1) << 4)) & 0xFFFF
            plsc.store_scatter(mask_v, [rt], ones16)

        @plsc.parallel_loop(0, _SEG, unroll=8,
                            carry=jnp.zeros((16,), jnp.int32))
        def _count(i, acc):
            m = mask_v[pl.ds(16 * i, 16)]
            return acc + jnp.where(m == 0, 1, 0)

        lane_cnt = _count
        lane_base = plsc.cumsum(lane_cnt) - lane_cnt

        cp_p.wait()

        @plsc.parallel_loop(0, _SEG, unroll=8, carry=lane_base)
        def _compact(i, run):
            m = mask_v[pl.ds(16 * i, 16)]
            pval = p_v[pl.ds(16 * i, 16)]
            keep = jnp.where(m == 0, 1, 0)
            valid = jnp.logical_and(m == 0, run < _B)
            dst = jnp.where(valid, run, _B)
            plsc.store_scatter(out_v, [dst], pval)
            return run + keep

        cp_io1 = pltpu.async_copy(it_v, io_hbm.at[pl.ds(0, _B)], s_a)
        cp_io2 = pltpu.async_copy(out_v.at[pl.ds(0, _B)],
                                  io_hbm.at[pl.ds(_B, _B)], s_b)

        cp_u.wait()
        iota16 = jnp.arange(16, dtype=jnp.int32)
        u16 = u_v[pl.ds(0, 16)]
        uvec = jnp.full((16,), jnp.sum(jnp.where(iota16 == 0, u16, 0)),
                        jnp.int32)

        @plsc.parallel_loop(0, 2 * _B // 16, unroll=8)
        def _ufill(i):
            uf_v[pl.ds(16 * i, 16)] = uvec

        cp_uo = pltpu.async_copy(uf_v, uo_hbm, s_c)

        cp_tg.wait()
        cp_to = pltpu.async_copy(tg_v, to_hbm, s_d)

        cp_io1.wait()
        cp_io2.wait()
        cp_uo.wait()
        cp_to.wait()


def kernel(user, item, target):
    return _negatives(user.astype(jnp.int32), item.astype(jnp.int32),
                      target.astype(jnp.float32), _SLOT2, _PREF_T)


# speedup vs baseline: 1.2091x; 1.0039x over previous
import functools

import jax
import jax.numpy as jnp
import numpy as np
from jax import lax
from jax.experimental import pallas as pl
from jax.experimental.pallas import tpu as pltpu
from jax.experimental.pallas import tpu_sc as plsc

_NUM_ITEMS = 100000
_B = 4096
_M = 2 * _B
_SEG = _M // 16


def _threefry_uniform_f32(seed: int, size: int) -> np.ndarray:
    def rotl(x, d):
        return ((x << np.uint32(d)) | (x >> np.uint32(32 - d))).astype(np.uint32)

    i64 = np.arange(size, dtype=np.uint64)
    x0 = (i64 >> np.uint64(32)).astype(np.uint32)
    x1 = (i64 & np.uint64(0xFFFFFFFF)).astype(np.uint32)
    k0 = np.uint32(seed >> 32)
    k1 = np.uint32(seed & 0xFFFFFFFF)
    ks = [k0, k1, np.uint32(k0 ^ k1 ^ np.uint32(0x1BD11BDA))]
    rotations = [(13, 15, 26, 6), (17, 29, 16, 24)]
    x0 = (x0 + ks[0]).astype(np.uint32)
    x1 = (x1 + ks[1]).astype(np.uint32)
    for i in range(5):
        for r in rotations[i % 2]:
            x0 = (x0 + x1).astype(np.uint32)
            x1 = rotl(x1, r)
            x1 = (x1 ^ x0).astype(np.uint32)
        x0 = (x0 + ks[(i + 1) % 3]).astype(np.uint32)
        x1 = (x1 + ks[(i + 2) % 3] + np.uint32(i + 1)).astype(np.uint32)
    bits = (x0 ^ x1).astype(np.uint32)
    mant = (bits >> np.uint32(9)) | np.uint32(0x3F800000)
    return mant.view(np.float32) - np.float32(1.0)


_scores = _threefry_uniform_f32(42, _NUM_ITEMS)
_order = np.argsort(-_scores, kind="stable").astype(np.int32)
_rank_np = np.empty((_NUM_ITEMS,), dtype=np.int32)
_rank_np[_order] = np.arange(_NUM_ITEMS, dtype=np.int32)
_slot_np = np.where(_rank_np < _M,
                    16 * (_rank_np & (_SEG - 1)) + (_rank_np >> 9),
                    _M).astype(np.uint32)
_SLOT2 = (_slot_np[0::2] | (_slot_np[1::2] << 16)).astype(np.int32)
_PREF_T = np.ascontiguousarray(
    _order[:_M].reshape(16, _SEG).T.reshape(-1))

_mesh = plsc.VectorSubcoreMesh(core_axis_name="c", subcore_axis_name="s",
                               num_cores=1)


@functools.partial(
    pl.kernel,
    out_type=(
        jax.ShapeDtypeStruct((2 * _B,), jnp.int32),
        jax.ShapeDtypeStruct((2 * _B,), jnp.int32),
        jax.ShapeDtypeStruct((2 * _B,), jnp.float32),
    ),
    mesh=_mesh,
    scratch_types=[
        pltpu.VMEM((_B,), jnp.int32),
        pltpu.VMEM((_NUM_ITEMS // 2,), jnp.int32),
        pltpu.VMEM((_M,), jnp.int32),
        pltpu.VMEM((_M + 256,), jnp.int32),
        pltpu.VMEM((_B + 16,), jnp.int32),
        pltpu.VMEM((16,), jnp.int32),
        pltpu.VMEM((2 * _B,), jnp.int32),
        pltpu.VMEM((2 * _B,), jnp.float32),
        pltpu.SemaphoreType.DMA,
        pltpu.SemaphoreType.DMA,
        pltpu.SemaphoreType.DMA,
        pltpu.SemaphoreType.DMA,
        pltpu.SemaphoreType.DMA,
    ],
    compiler_params=pltpu.CompilerParams(
        needs_layout_passes=False,
        disable_bounds_checks=True,
        disable_semaphore_checks=True,
    ),
)
def _negatives(user_hbm, item_hbm, target_hbm, slot_hbm, pref_hbm,
               uo_hbm, io_hbm, to_hbm,
               it_v, tbl_v, p_v, mask_v, out_v, u_v, uf_v, tg_v,
               s_a, s_b, s_c, s_d, s_e):
    c = lax.axis_index("c")
    s = lax.axis_index("s")

    @pl.when(jnp.logical_and(c == 0, s == 0))
    def _():
        cp_it = pltpu.async_copy(item_hbm, it_v, s_a)
        cp_tb = pltpu.async_copy(slot_hbm, tbl_v, s_b)
        cp_p = pltpu.async_copy(pref_hbm, p_v, s_c)
        cp_u = pltpu.async_copy(user_hbm.at[pl.ds(0, 16)], u_v, s_d)
        cp_tg = pltpu.async_copy(target_hbm, tg_v.at[pl.ds(0, _B)], s_e)

        zeros16 = jnp.zeros((16,), jnp.int32)
        ones16 = jnp.ones((16,), jnp.int32)
        fzeros16 = jnp.zeros((16,), jnp.float32)

        @plsc.parallel_loop(0, _B // 16, unroll=8)
        def _tzero(i):
            tg_v[pl.ds(_B + 16 * i, 16)] = fzeros16

        @plsc.parallel_loop(0, (_M + 256) // 16, unroll=8)
        def _zero(i):
            mask_v[pl.ds(16 * i, 16)] = zeros16

        cp_it.wait()
        cp_tb.wait()

        @plsc.parallel_loop(0, _B // 16, unroll=8)
        def _mark(i):
            idx = it_v[pl.ds(16 * i, 16)]
            word = plsc.load_gather(tbl_v, [idx >> 1])
            rt = (word >> ((idx & 1) << 4)) & 0xFFFF
            plsc.store_scatter(mask_v, [rt], ones16)

        @plsc.parallel_loop(0, _SEG, unroll=8,
                            carry=jnp.zeros((16,), jnp.int32))
        def _count(i, acc):
            m = mask_v[pl.ds(16 * i, 16)]
            return acc + jnp.where(m == 0, 1, 0)

        lane_cnt = _count
        lane_base = plsc.cumsum(lane_cnt) - lane_cnt

        cp_p.wait()

        @plsc.parallel_loop(0, _SEG, unroll=8, carry=lane_base)
        def _compact(i, run):
            m = mask_v[pl.ds(16 * i, 16)]
            pval = p_v[pl.ds(16 * i, 16)]
            keep = jnp.where(m == 0, 1, 0)
            valid = jnp.logical_and(m == 0, run < _B)
            dst = jnp.where(valid, run, _B)
            plsc.store_scatter(out_v, [dst], pval)
            return run + keep

        cp_io1 = pltpu.async_copy(it_v, io_hbm.at[pl.ds(0, _B)], s_a)
        cp_io2 = pltpu.async_copy(out_v.at[pl.ds(0, _B)],
                                  io_hbm.at[pl.ds(_B, _B)], s_b)

        cp_u.wait()
        iota16 = jnp.arange(16, dtype=jnp.int32)
        u16 = u_v[pl.ds(0, 16)]
        uvec = jnp.full((16,), jnp.sum(jnp.where(iota16 == 0, u16, 0)),
                        jnp.int32)

        @plsc.parallel_loop(0, 2 * _B // 16, unroll=8)
        def _ufill(i):
            uf_v[pl.ds(16 * i, 16)] = uvec

        cp_uo = pltpu.async_copy(uf_v, uo_hbm, s_c)

        cp_tg.wait()
        cp_to = pltpu.async_copy(tg_v, to_hbm, s_d)

        cp_io1.wait()
        cp_io2.wait()
        cp_uo.wait()
        cp_to.wait()


def kernel(user, item, target):
    return _negatives(user.astype(jnp.int32), item.astype(jnp.int32),
                      target.astype(jnp.float32), _SLOT2, _PREF_T)
